# Initial kernel scaffold; baseline (speedup 1.0000x reference)
import jax
import jax.numpy as jnp
from jax.experimental import pallas as pl


def kernel(f_atom, atom_pos, T_frag, frag_id, n_frag, frag_sizes):
    raise NotImplementedError("write your pallas kernel here")


# speedup vs baseline: 6.1890x; 6.1890x over previous
import functools

import jax
import jax.numpy as jnp
from jax import lax
from jax.experimental import pallas as pl
from jax.experimental.pallas import tpu as pltpu
from jax.experimental.pallas import tpu_sc as plsc

N_ATOM = 500000
N_FRAG = 25000

NC = 2
NS = 16
NW = NC * NS

B = 2000
GROUPS = B // 16
SCAT = 125
NSCAT = B // SCAT
CHUNK = 16000
NB_FULL = CHUNK // B
LAST_CHUNK = N_ATOM - (NW - 1) * CHUNK
NB_LAST = LAST_CHUNK // B

ACC_ROWS = 25088
ZROWS = ACC_ROWS // NS
FROWS = ACC_ROWS // NW
FGROUPS = FROWS // 16

_PARAMS = dict(
    compiler_params=pltpu.CompilerParams(
        needs_layout_passes=False, use_tc_tiling_on_sc=False),
)


def _mesh():
    return plsc.VectorSubcoreMesh(
        core_axis_name="c", subcore_axis_name="s", num_cores=NC,
        num_subcores=NS)


def _seg_accumulate(f_atom, atom_pos, fid2):
    @functools.partial(
        pl.kernel,
        out_type=jax.ShapeDtypeStruct((NC, ACC_ROWS, 16), jnp.float32),
        mesh=_mesh(),
        scratch_types=[
            pltpu.VMEM((B, 3), jnp.float32),
            pltpu.VMEM((B, 3), jnp.float32),
            pltpu.VMEM((NSCAT, SCAT), jnp.int32),
            pltpu.VMEM((B, 16), jnp.float32),
            pltpu.VMEM_SHARED((ACC_ROWS, 16), jnp.float32),
        ],
        **_PARAMS,
    )
    def k1(f_hbm, p_hbm, id_hbm, out_hbm, fbuf, pbuf, idbuf, feat, acc):
        c = lax.axis_index("c")
        s = lax.axis_index("s")
        wid = c * NS + s

        iota = lax.iota(jnp.int32, 16)
        c0 = jnp.zeros((16,), jnp.int32)
        zeros = jnp.zeros((16,), jnp.float32)
        ones = jnp.ones((16,), jnp.float32)

        def zrow(i, _):
            feat[i, :] = zeros
            return 0
        lax.fori_loop(0, ZROWS, zrow, 0)
        zbase = pl.multiple_of(s * ZROWS, ZROWS)
        pltpu.sync_copy(feat.at[pl.ds(0, ZROWS)], acc.at[pl.ds(zbase, ZROWS)])

        def crow(g, _):
            plsc.store_scatter(feat, [g * 16 + iota, c0], ones)
            return 0
        lax.fori_loop(0, GROUPS, crow, 0)
        plsc.subcore_barrier()

        nb = jnp.where(wid == NW - 1, NB_LAST, NB_FULL)

        def batch(b, _):
            abase = pl.multiple_of(wid * CHUNK + b * B, B)
            idrow = pl.multiple_of((wid * CHUNK + b * B) // SCAT, NSCAT)
            pltpu.sync_copy(f_hbm.at[pl.ds(abase, B)], fbuf)
            pltpu.sync_copy(p_hbm.at[pl.ds(abase, B)], pbuf)
            pltpu.sync_copy(id_hbm.at[pl.ds(idrow, NSCAT)], idbuf)

            def group(g, _):
                rows = g * 16 + iota
                fx = plsc.load_gather(fbuf, [rows, c0])
                fy = plsc.load_gather(fbuf, [rows, c0 + 1])
                fz = plsc.load_gather(fbuf, [rows, c0 + 2])
                px = plsc.load_gather(pbuf, [rows, c0])
                py = plsc.load_gather(pbuf, [rows, c0 + 1])
                pz = plsc.load_gather(pbuf, [rows, c0 + 2])

                vals = (
                    fx, fy, fz,
                    px, py, pz,
                    py * fz - pz * fy,
                    pz * fx - px * fz,
                    px * fy - py * fx,
                    px * px, py * py, pz * pz,
                    px * py, px * pz, py * pz,
                )
                for k, v in enumerate(vals):
                    plsc.store_scatter(feat, [rows, c0 + (k + 1)], v)
                return 0
            lax.fori_loop(0, GROUPS, group, 0)

            def scat(j, _):
                pltpu.sync_copy(feat.at[pl.ds(j * SCAT, SCAT)],
                                acc.at[idbuf.at[j]], add=True)
                return 0
            lax.fori_loop(0, NSCAT, scat, 0)
            return 0
        lax.fori_loop(0, nb, batch, 0)

        plsc.subcore_barrier()
        obase = pl.multiple_of(s * ZROWS, ZROWS)
        pltpu.sync_copy(acc.at[pl.ds(obase, ZROWS)],
                        out_hbm.at[c].at[pl.ds(obase, ZROWS)])

    return k1(f_atom, atom_pos, fid2)


def _postprocess(part, T_pad, fsz_pad):
    @functools.partial(
        pl.kernel,
        out_type=(jax.ShapeDtypeStruct((ACC_ROWS, 3), jnp.float32),
                  jax.ShapeDtypeStruct((ACC_ROWS, 3), jnp.float32)),
        mesh=_mesh(),
        scratch_types=[
            pltpu.VMEM((FROWS, 16), jnp.float32),
            pltpu.VMEM((FROWS, 16), jnp.float32),
            pltpu.VMEM((FROWS, 3), jnp.float32),
            pltpu.VMEM((FROWS,), jnp.int32),
            pltpu.VMEM((FROWS, 3), jnp.float32),
            pltpu.VMEM((FROWS, 3), jnp.float32),
        ],
        **_PARAMS,
    )
    def k2(part_hbm, t_hbm, sz_hbm, v_hbm, o_hbm, p0, p1, tbuf, szbuf,
           vstg, ostg):
        c = lax.axis_index("c")
        s = lax.axis_index("s")
        wid = c * NS + s
        r0 = pl.multiple_of(wid * FROWS, FROWS)

        pltpu.sync_copy(part_hbm.at[0].at[pl.ds(r0, FROWS)], p0)
        pltpu.sync_copy(part_hbm.at[1].at[pl.ds(r0, FROWS)], p1)
        pltpu.sync_copy(t_hbm.at[pl.ds(r0, FROWS)], tbuf)
        pltpu.sync_copy(sz_hbm.at[pl.ds(r0, FROWS)], szbuf)

        def addrow(i, _):
            p0[i, :] = p0[i, :] + p1[i, :]
            return 0
        lax.fori_loop(0, FROWS, addrow, 0)

        iota = lax.iota(jnp.int32, 16)
        c0 = jnp.zeros((16,), jnp.int32)

        def group(g, _):
            o = g * 16
            rows = o + iota

            def F(k):
                return plsc.load_gather(p0, [rows, c0 + k])

            cnt = F(0)
            sfx, sfy, sfz = F(1), F(2), F(3)
            spx, spy, spz = F(4), F(5), F(6)
            cpx, cpy, cpz = F(7), F(8), F(9)
            sxx, syy, szz = F(10), F(11), F(12)
            sxy, sxz, syz = F(13), F(14), F(15)

            tx = plsc.load_gather(tbuf, [rows, c0])
            ty = plsc.load_gather(tbuf, [rows, c0 + 1])
            tz = plsc.load_gather(tbuf, [rows, c0 + 2])
            sz = szbuf[pl.ds(o, 16)]

            denom = jnp.maximum(cnt, 1.0)
            vx = sfx / denom
            vy = sfy / denom
            vz = sfz / denom

            tqx = cpx - (ty * sfz - tz * sfy)
            tqy = cpy - (tz * sfx - tx * sfz)
            tqz = cpz - (tx * sfy - ty * sfx)

            mxx = sxx - 2.0 * tx * spx + cnt * tx * tx
            myy = syy - 2.0 * ty * spy + cnt * ty * ty
            mzz = szz - 2.0 * tz * spz + cnt * tz * tz
            mxy = sxy - tx * spy - ty * spx + cnt * tx * ty
            mxz = sxz - tx * spz - tz * spx + cnt * tx * tz
            myz = syz - ty * spz - tz * spy + cnt * ty * tz

            tr = mxx + myy + mzz
            a = tr - mxx + 1e-4
            bb = tr - myy + 1e-4
            cc = tr - mzz + 1e-4
            d = -mxy
            e = -mxz
            f = -myz

            c11 = bb * cc - f * f
            c12 = e * f - d * cc
            c13 = d * f - bb * e
            c22 = a * cc - e * e
            c23 = d * e - a * f
            c33 = a * bb - d * d
            det = a * c11 + d * c12 + e * c13
            rdet = 1.0 / det

            ox = (c11 * tqx + c12 * tqy + c13 * tqz) * rdet
            oy = (c12 * tqx + c22 * tqy + c23 * tqz) * rdet
            oz = (c13 * tqx + c23 * tqy + c33 * tqz) * rdet

            single = sz <= 1
            zf = jnp.zeros((16,), jnp.float32)
            ox = jnp.where(single, zf, ox)
            oy = jnp.where(single, zf, oy)
            oz = jnp.where(single, zf, oz)

            plsc.store_scatter(vstg, [rows, c0], vx)
            plsc.store_scatter(vstg, [rows, c0 + 1], vy)
            plsc.store_scatter(vstg, [rows, c0 + 2], vz)
            plsc.store_scatter(ostg, [rows, c0], ox)
            plsc.store_scatter(ostg, [rows, c0 + 1], oy)
            plsc.store_scatter(ostg, [rows, c0 + 2], oz)
            return 0
        lax.fori_loop(0, FGROUPS, group, 0)

        pltpu.sync_copy(vstg, v_hbm.at[pl.ds(r0, FROWS)])
        pltpu.sync_copy(ostg, o_hbm.at[pl.ds(r0, FROWS)])

    return k2(part, T_pad, fsz_pad)


def kernel(f_atom, atom_pos, T_frag, frag_id, n_frag, frag_sizes):
    del n_frag
    fid = frag_id.astype(jnp.int32).reshape(N_ATOM // SCAT, SCAT)
    part = _seg_accumulate(f_atom, atom_pos, fid)

    pad = ACC_ROWS - N_FRAG
    T_pad = jnp.pad(T_frag, ((0, pad), (0, 0)))
    fsz_pad = jnp.pad(frag_sizes.astype(jnp.int32), (0, pad))
    v_pad, o_pad = _postprocess(part, T_pad, fsz_pad)
    return v_pad[:N_FRAG], o_pad[:N_FRAG]


# speedup vs baseline: 6.7518x; 1.0909x over previous
import functools

import jax
import jax.numpy as jnp
from jax import lax
from jax.experimental import pallas as pl
from jax.experimental.pallas import tpu as pltpu
from jax.experimental.pallas import tpu_sc as plsc

N_ATOM = 500000
N_FRAG = 25000

NC = 2
NS = 16
NW = NC * NS

B = 2000
GROUPS = B // 16
SCAT = 125
NSCAT = B // SCAT
CHUNK = 16000
NB_FULL = CHUNK // B
LAST_CHUNK = N_ATOM - (NW - 1) * CHUNK
NB_LAST = LAST_CHUNK // B

ACC_ROWS = 25088
ZROWS = ACC_ROWS // NS
FROWS = ACC_ROWS // NW
FGROUPS = FROWS // 16

_PARAMS = dict(
    compiler_params=pltpu.CompilerParams(
        needs_layout_passes=False, use_tc_tiling_on_sc=False),
)


def _mesh():
    return plsc.VectorSubcoreMesh(
        core_axis_name="c", subcore_axis_name="s", num_cores=NC,
        num_subcores=NS)


def _seg_accumulate(f_flat, p_flat, fid2):
    @functools.partial(
        pl.kernel,
        out_type=jax.ShapeDtypeStruct((NC, ACC_ROWS, 16), jnp.float32),
        mesh=_mesh(),
        scratch_types=[
            pltpu.VMEM((3 * B,), jnp.float32),
            pltpu.VMEM((3 * B,), jnp.float32),
            pltpu.VMEM((NSCAT, SCAT), jnp.int32),
            pltpu.VMEM((B, 16), jnp.float32),
            pltpu.VMEM_SHARED((ACC_ROWS, 16), jnp.float32),
        ],
        **_PARAMS,
    )
    def k1(f_hbm, p_hbm, id_hbm, out_hbm, fbuf, pbuf, idbuf, feat, acc):
        c = lax.axis_index("c")
        s = lax.axis_index("s")
        wid = c * NS + s

        iota = lax.iota(jnp.int32, 16)
        iota3 = iota * 3
        c0 = jnp.zeros((16,), jnp.int32)
        zeros = jnp.zeros((16,), jnp.float32)
        ones = jnp.ones((16,), jnp.float32)

        def zrow(i, _):
            feat[i, :] = zeros
            return 0
        lax.fori_loop(0, ZROWS, zrow, 0)
        zbase = pl.multiple_of(s * ZROWS, ZROWS)
        pltpu.sync_copy(feat.at[pl.ds(0, ZROWS)], acc.at[pl.ds(zbase, ZROWS)])

        def crow(g, _):
            plsc.store_scatter(feat, [g * 16 + iota, c0], ones)
            return 0
        lax.fori_loop(0, GROUPS, crow, 0)
        plsc.subcore_barrier()

        nb = jnp.where(wid == NW - 1, NB_LAST, NB_FULL)

        def batch(b, _):
            a3 = pl.multiple_of((wid * CHUNK + b * B) * 3, 3 * B)
            idrow = pl.multiple_of((wid * CHUNK + b * B) // SCAT, NSCAT)
            pltpu.sync_copy(f_hbm.at[pl.ds(a3, 3 * B)], fbuf)
            pltpu.sync_copy(p_hbm.at[pl.ds(a3, 3 * B)], pbuf)
            pltpu.sync_copy(id_hbm.at[pl.ds(idrow, NSCAT)], idbuf)

            def group(g, _):
                rows = g * 16 + iota
                o3 = g * 48 + iota3
                fx = plsc.load_gather(fbuf, [o3])
                fy = plsc.load_gather(fbuf, [o3 + 1])
                fz = plsc.load_gather(fbuf, [o3 + 2])
                px = plsc.load_gather(pbuf, [o3])
                py = plsc.load_gather(pbuf, [o3 + 1])
                pz = plsc.load_gather(pbuf, [o3 + 2])

                vals = (
                    fx, fy, fz,
                    px, py, pz,
                    py * fz - pz * fy,
                    pz * fx - px * fz,
                    px * fy - py * fx,
                    px * px, py * py, pz * pz,
                    px * py, px * pz, py * pz,
                )
                for k, v in enumerate(vals):
                    plsc.store_scatter(feat, [rows, c0 + (k + 1)], v)
                return 0
            lax.fori_loop(0, GROUPS, group, 0)

            def scat(j, _):
                pltpu.sync_copy(feat.at[pl.ds(j * SCAT, SCAT)],
                                acc.at[idbuf.at[j]], add=True)
                return 0
            lax.fori_loop(0, NSCAT, scat, 0)
            return 0
        lax.fori_loop(0, nb, batch, 0)

        plsc.subcore_barrier()
        obase = pl.multiple_of(s * ZROWS, ZROWS)
        pltpu.sync_copy(acc.at[pl.ds(obase, ZROWS)],
                        out_hbm.at[c].at[pl.ds(obase, ZROWS)])

    return k1(f_flat, p_flat, fid2)


def _postprocess(part, T_flat, fsz_pad):
    @functools.partial(
        pl.kernel,
        out_type=(jax.ShapeDtypeStruct((ACC_ROWS * 3,), jnp.float32),
                  jax.ShapeDtypeStruct((ACC_ROWS * 3,), jnp.float32)),
        mesh=_mesh(),
        scratch_types=[
            pltpu.VMEM((FROWS, 16), jnp.float32),
            pltpu.VMEM((FROWS, 16), jnp.float32),
            pltpu.VMEM((FROWS * 3,), jnp.float32),
            pltpu.VMEM((FROWS,), jnp.int32),
            pltpu.VMEM((FROWS * 3,), jnp.float32),
            pltpu.VMEM((FROWS * 3,), jnp.float32),
        ],
        **_PARAMS,
    )
    def k2(part_hbm, t_hbm, sz_hbm, v_hbm, o_hbm, p0, p1, tbuf, szbuf,
           vstg, ostg):
        c = lax.axis_index("c")
        s = lax.axis_index("s")
        wid = c * NS + s
        r0 = pl.multiple_of(wid * FROWS, FROWS)
        r3 = pl.multiple_of(wid * (FROWS * 3), FROWS * 3)

        pltpu.sync_copy(part_hbm.at[0].at[pl.ds(r0, FROWS)], p0)
        pltpu.sync_copy(part_hbm.at[1].at[pl.ds(r0, FROWS)], p1)
        pltpu.sync_copy(t_hbm.at[pl.ds(r3, FROWS * 3)], tbuf)
        pltpu.sync_copy(sz_hbm.at[pl.ds(r0, FROWS)], szbuf)

        def addrow(i, _):
            p0[i, :] = p0[i, :] + p1[i, :]
            return 0
        lax.fori_loop(0, FROWS, addrow, 0)

        iota = lax.iota(jnp.int32, 16)
        iota3 = iota * 3
        c0 = jnp.zeros((16,), jnp.int32)

        def group(g, _):
            o = g * 16
            rows = o + iota
            o3 = g * 48 + iota3

            def F(k):
                return plsc.load_gather(p0, [rows, c0 + k])

            cnt = F(0)
            sfx, sfy, sfz = F(1), F(2), F(3)
            spx, spy, spz = F(4), F(5), F(6)
            cpx, cpy, cpz = F(7), F(8), F(9)
            sxx, syy, szz = F(10), F(11), F(12)
            sxy, sxz, syz = F(13), F(14), F(15)

            tx = plsc.load_gather(tbuf, [o3])
            ty = plsc.load_gather(tbuf, [o3 + 1])
            tz = plsc.load_gather(tbuf, [o3 + 2])
            sz = szbuf[pl.ds(o, 16)]

            denom = jnp.maximum(cnt, 1.0)
            vx = sfx / denom
            vy = sfy / denom
            vz = sfz / denom

            tqx = cpx - (ty * sfz - tz * sfy)
            tqy = cpy - (tz * sfx - tx * sfz)
            tqz = cpz - (tx * sfy - ty * sfx)

            mxx = sxx - 2.0 * tx * spx + cnt * tx * tx
            myy = syy - 2.0 * ty * spy + cnt * ty * ty
            mzz = szz - 2.0 * tz * spz + cnt * tz * tz
            mxy = sxy - tx * spy - ty * spx + cnt * tx * ty
            mxz = sxz - tx * spz - tz * spx + cnt * tx * tz
            myz = syz - ty * spz - tz * spy + cnt * ty * tz

            tr = mxx + myy + mzz
            a = tr - mxx + 1e-4
            bb = tr - myy + 1e-4
            cc = tr - mzz + 1e-4
            d = -mxy
            e = -mxz
            f = -myz

            c11 = bb * cc - f * f
            c12 = e * f - d * cc
            c13 = d * f - bb * e
            c22 = a * cc - e * e
            c23 = d * e - a * f
            c33 = a * bb - d * d
            det = a * c11 + d * c12 + e * c13
            rdet = 1.0 / det

            ox = (c11 * tqx + c12 * tqy + c13 * tqz) * rdet
            oy = (c12 * tqx + c22 * tqy + c23 * tqz) * rdet
            oz = (c13 * tqx + c23 * tqy + c33 * tqz) * rdet

            single = sz <= 1
            zf = jnp.zeros((16,), jnp.float32)
            ox = jnp.where(single, zf, ox)
            oy = jnp.where(single, zf, oy)
            oz = jnp.where(single, zf, oz)

            plsc.store_scatter(vstg, [o3], vx)
            plsc.store_scatter(vstg, [o3 + 1], vy)
            plsc.store_scatter(vstg, [o3 + 2], vz)
            plsc.store_scatter(ostg, [o3], ox)
            plsc.store_scatter(ostg, [o3 + 1], oy)
            plsc.store_scatter(ostg, [o3 + 2], oz)
            return 0
        lax.fori_loop(0, FGROUPS, group, 0)

        pltpu.sync_copy(vstg, v_hbm.at[pl.ds(r3, FROWS * 3)])
        pltpu.sync_copy(ostg, o_hbm.at[pl.ds(r3, FROWS * 3)])

    return k2(part, T_flat, fsz_pad)


def kernel(f_atom, atom_pos, T_frag, frag_id, n_frag, frag_sizes):
    del n_frag
    fid = frag_id.astype(jnp.int32).reshape(N_ATOM // SCAT, SCAT)
    part = _seg_accumulate(f_atom.reshape(-1), atom_pos.reshape(-1), fid)

    pad = ACC_ROWS - N_FRAG
    T_flat = jnp.pad(T_frag, ((0, pad), (0, 0))).reshape(-1)
    fsz_pad = jnp.pad(frag_sizes.astype(jnp.int32), (0, pad))
    v_flat, o_flat = _postprocess(part, T_flat, fsz_pad)
    return (v_flat.reshape(ACC_ROWS, 3)[:N_FRAG],
            o_flat.reshape(ACC_ROWS, 3)[:N_FRAG])
